# baseline (speedup 1.0000x reference)
import jax
import jax.numpy as jnp
from jax import lax
from jax.experimental import pallas as pl
from jax.experimental.pallas import tpu as pltpu
from jax.experimental.pallas import tpu_sc as plsc

B, L, D, K = 256, 128, 128, 32
NC, NS = 2, 16
NW = NC * NS
B_SC = 64
B_TC = B - B_SC
BPW = B_SC // NW
CH = D // 16


def _sc_agg_body(feats2_hbm, head_hbm, out_hbm,
                 fv0, fv1, fv2, hv0, hv1, hv2, pv0, pv1, cv0, cv1,
                 gi0, gi1, ones_v, zcnt_v, rdeg_v, crd_v,
                 acc_sh, cnt_sh, *sems):
    cid = lax.axis_index("c")
    sid = lax.axis_index("s")
    wid = sid * NC + cid
    base = wid * BPW

    fvs, hvs = [fv0, fv1, fv2], [hv0, hv1, hv2]
    pvs, cvs, gis = [pv0, pv1], [cv0, cv1], [gi0, gi1]
    (sA0, sA1, sA2, sH0, sH1, sH2, sB0, sB1, sC0, sC1,
     sD0, sD1, sP0, sP1, sF0, sF1, sF2) = sems
    sAs, sHs, sFs = [sA0, sA1, sA2], [sH0, sH1, sH2], [sF0, sF1, sF2]
    sPs = [sP0, sP1]

    iota16 = lax.iota(jnp.int32, 16)

    def init_const(t, _):
        ones_v[t, :] = jnp.ones((16,), jnp.float32)
        zcnt_v[t, :] = jnp.zeros((16,), jnp.float32)
        return 0
    lax.fori_loop(0, L, init_const, 0)

    def issue_load(q):
        r = q % 3
        a = pltpu.async_copy(feats2_hbm.at[pl.ds((base + q) * L, L)],
                             fvs[r], sAs[r])
        h = pltpu.async_copy(head_hbm.at[base + q], hvs[r], sHs[r])
        return a, h

    def issue_par(q):
        r, p = q % 3, q % 2
        off = (base + q) * L
        for t in range(L // 16):
            gis[p][pl.ds(t * 16, 16)] = hvs[r][pl.ds(t * 16, 16)] + off
        return pltpu.async_copy(feats2_hbm.at[gis[p]], pvs[p], sPs[p])

    def issue_init(q):
        r = q % 3
        b1 = pltpu.async_copy(fvs[r], acc_sh.at[sid], sB0)
        b2 = pltpu.async_copy(zcnt_v, cnt_sh.at[sid], sB1)
        return b1, b2

    def issue_scatter(q):
        r = q % 3
        c1 = pltpu.async_copy(fvs[r], acc_sh.at[sid].at[hvs[r]], sC0,
                              add=True)
        c2 = pltpu.async_copy(ones_v, cnt_sh.at[sid].at[hvs[r]], sC1,
                              add=True)
        return c1, c2

    def issue_readback(q):
        r, p = q % 3, q % 2
        d1 = pltpu.async_copy(acc_sh.at[sid], fvs[r], sD0)
        d2 = pltpu.async_copy(cnt_sh.at[sid], cvs[p], sD1)
        return d1, d2

    def coef_phase(q):
        r, p = q % 3, q % 2
        for t in range(L // 16):
            h16 = hvs[r][pl.ds(t * 16, 16)]
            i16 = iota16 + t * 16
            hh = plsc.load_gather(hvs[r], [h16])
            pf = (h16 != i16).astype(jnp.int32)
            m = jnp.where(hh == i16, pf, 0)
            coef = (2 * pf - m - 1).astype(jnp.float32)
            cnt16 = plsc.load_gather(cvs[p], [i16, jnp.zeros((16,), jnp.int32)])
            rdeg = 1.0 / (cnt16 + coef + 1.0)
            rdeg_v[pl.ds(t * 16, 16)] = rdeg
            crd_v[pl.ds(t * 16, 16)] = coef * rdeg

    def finalize_rows(q, lo, hi):
        r, p = q % 3, q % 2
        av, pv = fvs[r], pvs[p]

        def row2(k, _):
            for u in range(2):
                i = k * 2 + lo + u
                isplat = jnp.full((16,), i, jnp.int32)
                rb = plsc.load_gather(rdeg_v, [isplat])
                cb = plsc.load_gather(crd_v, [isplat])
                for j in range(CH):
                    sl = pl.ds(j * 16, 16)
                    av[i, sl] = av[i, sl] * rb + pv[i, sl] * cb
            return 0
        lax.fori_loop(0, (hi - lo) // 2, row2, 0)

    ad = [None] * BPW
    hd = [None] * BPW
    fd = [None] * BPW
    pd = [None] * BPW
    ad[0], hd[0] = issue_load(0)
    if BPW > 1:
        ad[1], hd[1] = issue_load(1)
    ad[0].wait(); hd[0].wait()
    b1, b2 = issue_init(0)
    pd[0] = issue_par(0)
    b1.wait(); b2.wait()
    c1, c2 = issue_scatter(0)
    c1.wait(); c2.wait()
    d1, d2 = issue_readback(0)
    d1.wait(); d2.wait()

    for q in range(BPW):
        nxt = q + 1 < BPW
        if nxt:
            ad[q + 1].wait(); hd[q + 1].wait()
            b1, b2 = issue_init(q + 1)
            pd[q + 1] = issue_par(q + 1)
        coef_phase(q)
        if nxt:
            b1.wait(); b2.wait()
            c1, c2 = issue_scatter(q + 1)
        if q + 2 < BPW:
            if q >= 1:
                fd[q - 1].wait()
            ad[q + 2], hd[q + 2] = issue_load(q + 2)
        pd[q].wait()
        finalize_rows(q, 0, L // 2)
        if nxt:
            c1.wait(); c2.wait()
            d1, d2 = issue_readback(q + 1)
        finalize_rows(q, L // 2, L)
        fd[q] = pltpu.async_copy(fvs[q % 3],
                                 out_hbm.at[pl.ds((base + q) * L, L)],
                                 sFs[q % 3])
        if nxt:
            d1.wait(); d2.wait()

    for q in range(max(0, BPW - 2), BPW):
        fd[q].wait()


def _sc_agg(feats2, head):
    mesh = plsc.VectorSubcoreMesh(core_axis_name="c", subcore_axis_name="s")
    return pl.kernel(
        _sc_agg_body,
        out_type=jax.ShapeDtypeStruct((B_SC * L, D), jnp.float32),
        mesh=mesh,
        compiler_params=pltpu.CompilerParams(needs_layout_passes=False,
                                             use_tc_tiling_on_sc=False),
        scratch_types=[
            pltpu.VMEM((L, D), jnp.float32),
            pltpu.VMEM((L, D), jnp.float32),
            pltpu.VMEM((L, D), jnp.float32),
            pltpu.VMEM((L,), jnp.int32),
            pltpu.VMEM((L,), jnp.int32),
            pltpu.VMEM((L,), jnp.int32),
            pltpu.VMEM((L, D), jnp.float32),
            pltpu.VMEM((L, D), jnp.float32),
            pltpu.VMEM((L, 16), jnp.float32),
            pltpu.VMEM((L, 16), jnp.float32),
            pltpu.VMEM((L,), jnp.int32),
            pltpu.VMEM((L,), jnp.int32),
            pltpu.VMEM((L, 16), jnp.float32),
            pltpu.VMEM((L, 16), jnp.float32),
            pltpu.VMEM((L,), jnp.float32),
            pltpu.VMEM((L,), jnp.float32),
            pltpu.VMEM_SHARED((NS, L, D), jnp.float32),
            pltpu.VMEM_SHARED((NS, L, 16), jnp.float32),
        ] + [pltpu.SemaphoreType.DMA] * 17,
    )(feats2, head)


GB = 16
GA = 16


def _tail(x, w_ref, b_ref, c_ref):
    h = jnp.dot(x, w_ref[...], preferred_element_type=jnp.float32) + b_ref[...]
    h = jnp.maximum(h, 0.0).astype(jnp.bfloat16)
    s = lax.dot_general(h, c_ref[...], (((1,), (1,)), ((), ())),
                        preferred_element_type=jnp.float32)
    mx = jnp.max(s, axis=-1, keepdims=True)
    e = jnp.exp(s - mx)
    return e / jnp.sum(e, axis=-1, keepdims=True)


def _tc_tail_body(x_ref, w_ref, b_ref, c_ref, o_ref):
    x = x_ref[...].astype(jnp.bfloat16)
    o_ref[...] = _tail(x, w_ref, b_ref, c_ref).reshape(GB, L, K)


def _tc_tail(aggn2, W_gnn, b_gnn, centroids):
    return pl.pallas_call(
        _tc_tail_body,
        grid=(B_SC // GB,),
        in_specs=[
            pl.BlockSpec((GB * L, D), lambda i: (i, 0)),
            pl.BlockSpec((D, D), lambda i: (0, 0)),
            pl.BlockSpec((1, D), lambda i: (0, 0)),
            pl.BlockSpec((K, D), lambda i: (0, 0)),
        ],
        out_specs=pl.BlockSpec((GB, L, K), lambda i: (i, 0, 0)),
        out_shape=jax.ShapeDtypeStruct((B, L, K), jnp.float32),
    )(aggn2, W_gnn.astype(jnp.bfloat16), b_gnn.reshape(1, D),
      centroids.astype(jnp.bfloat16))


def _tc_full_body(feats_ref, head_ref, w_ref, b_ref, c_ref, part_ref, o_ref):
    del part_ref
    ii = lax.broadcasted_iota(jnp.int32, (L, L), 0)
    jj = lax.broadcasted_iota(jnp.int32, (L, L), 1)
    eye = (ii == jj)
    aggs = []
    for s in range(GA):
        h_row = head_ref[s].reshape(1, L)
        h_col = h_row.reshape(L, 1)
        oh = (h_col == jj)
        ohT = (h_row == ii)
        adj = jnp.minimum(oh.astype(jnp.float32) + ohT.astype(jnp.float32)
                          + eye.astype(jnp.float32), 1.0)
        rdeg = 1.0 / jnp.maximum(jnp.sum(adj, axis=1, keepdims=True), 1.0)
        fs = feats_ref[s].astype(jnp.bfloat16)
        agg = jnp.dot(adj.astype(jnp.bfloat16), fs,
                      preferred_element_type=jnp.float32)
        aggs.append(agg * rdeg)
    x = jnp.concatenate(aggs, axis=0).astype(jnp.bfloat16)
    o_ref[...] = _tail(x, w_ref, b_ref, c_ref).reshape(GA, L, K)


def _tc_full(feats, head, W_gnn, b_gnn, centroids, partial_out):
    off = B_SC // GA
    return pl.pallas_call(
        _tc_full_body,
        grid=(B_TC // GA,),
        in_specs=[
            pl.BlockSpec((GA, L, D), lambda i: (i + off, 0, 0)),
            pl.BlockSpec((GA, L), lambda i: (i + off, 0)),
            pl.BlockSpec((D, D), lambda i: (0, 0)),
            pl.BlockSpec((1, D), lambda i: (0, 0)),
            pl.BlockSpec((K, D), lambda i: (0, 0)),
            pl.BlockSpec((GA, L, K), lambda i: (0, 0, 0)),
        ],
        out_specs=pl.BlockSpec((GA, L, K), lambda i: (i + off, 0, 0)),
        out_shape=jax.ShapeDtypeStruct((B, L, K), jnp.float32),
        input_output_aliases={5: 0},
    )(feats, head, W_gnn.astype(jnp.bfloat16), b_gnn.reshape(1, D),
      centroids.astype(jnp.bfloat16), partial_out)


def kernel(feats, tokens, aspect, pos, post, head, deprel, sen_len, adk,
           pos_mask, word_mask, aspect_pos_start, aspect_pos_end,
           plain_text, text_list, W_gnn, b_gnn, centroids):
    h32 = head.astype(jnp.int32)
    aggn2 = _sc_agg(feats.reshape(B * L, D), h32)
    out_sc = _tc_tail(aggn2, W_gnn, b_gnn, centroids)
    return _tc_full(feats, h32, W_gnn, b_gnn, centroids, out_sc)


# speedup vs baseline: 1.3617x; 1.3617x over previous
import jax
import jax.numpy as jnp
from jax import lax
from jax.experimental import pallas as pl
from jax.experimental.pallas import tpu as pltpu
from jax.experimental.pallas import tpu_sc as plsc

B, L, D, K = 256, 128, 128, 32
NC, NS = 2, 16
NW = NC * NS
B_SC = 64
B_TC = B - B_SC
BPW = B_SC // NW
CH = D // 16


def _sc_agg_body(feats2_hbm, head_hbm, out_hbm,
                 fv0, fv1, fv2, hv0, hv1, hv2, pv0, pv1, cv0, cv1,
                 gi0, gi1, ones_v, zcnt_v, rdeg_v, crd_v,
                 acc_sh, cnt_sh, *sems):
    cid = lax.axis_index("c")
    sid = lax.axis_index("s")
    wid = sid * NC + cid
    base = wid * BPW

    fvs, hvs = [fv0, fv1, fv2], [hv0, hv1, hv2]
    pvs, cvs, gis = [pv0, pv1], [cv0, cv1], [gi0, gi1]
    (sA0, sA1, sA2, sH0, sH1, sH2, sB0, sB1, sC0, sC1,
     sD0, sD1, sP0, sP1, sF0, sF1, sF2) = sems
    sAs, sHs, sFs = [sA0, sA1, sA2], [sH0, sH1, sH2], [sF0, sF1, sF2]
    sPs = [sP0, sP1]

    iota16 = lax.iota(jnp.int32, 16)

    def init_const(t, _):
        ones_v[t, :] = jnp.ones((16,), jnp.float32)
        zcnt_v[t, :] = jnp.zeros((16,), jnp.float32)
        return 0
    lax.fori_loop(0, L, init_const, 0)

    def issue_load(q):
        r = q % 3
        a = pltpu.async_copy(feats2_hbm.at[pl.ds((base + q) * L, L)],
                             fvs[r], sAs[r])
        h = pltpu.async_copy(head_hbm.at[base + q], hvs[r], sHs[r])
        return a, h

    def issue_par(q):
        r, p = q % 3, q % 2
        off = (base + q) * L
        for t in range(L // 16):
            gis[p][pl.ds(t * 16, 16)] = hvs[r][pl.ds(t * 16, 16)] + off
        return pltpu.async_copy(feats2_hbm.at[gis[p]], pvs[p], sPs[p])

    def issue_init(q):
        r = q % 3
        b1 = pltpu.async_copy(fvs[r], acc_sh.at[sid], sB0)
        b2 = pltpu.async_copy(zcnt_v, cnt_sh.at[sid], sB1)
        return b1, b2

    def issue_scatter(q):
        r = q % 3
        c1 = pltpu.async_copy(fvs[r], acc_sh.at[sid].at[hvs[r]], sC0,
                              add=True)
        c2 = pltpu.async_copy(ones_v, cnt_sh.at[sid].at[hvs[r]], sC1,
                              add=True)
        return c1, c2

    def issue_readback(q):
        r, p = q % 3, q % 2
        d1 = pltpu.async_copy(acc_sh.at[sid], fvs[r], sD0)
        d2 = pltpu.async_copy(cnt_sh.at[sid], cvs[p], sD1)
        return d1, d2

    def coef_phase(q):
        r, p = q % 3, q % 2
        for t in range(L // 16):
            h16 = hvs[r][pl.ds(t * 16, 16)]
            i16 = iota16 + t * 16
            hh = plsc.load_gather(hvs[r], [h16])
            pf = (h16 != i16).astype(jnp.int32)
            m = jnp.where(hh == i16, pf, 0)
            coef = (2 * pf - m - 1).astype(jnp.float32)
            cnt16 = plsc.load_gather(cvs[p], [i16, jnp.zeros((16,), jnp.int32)])
            rdeg = 1.0 / (cnt16 + coef + 1.0)
            rdeg_v[pl.ds(t * 16, 16)] = rdeg
            crd_v[pl.ds(t * 16, 16)] = coef * rdeg

    def finalize_rows(q, lo, hi):
        r, p = q % 3, q % 2
        av, pv = fvs[r], pvs[p]

        def row2(k, _):
            for u in range(2):
                i = k * 2 + lo + u
                isplat = jnp.full((16,), i, jnp.int32)
                rb = plsc.load_gather(rdeg_v, [isplat])
                cb = plsc.load_gather(crd_v, [isplat])
                for j in range(CH):
                    sl = pl.ds(j * 16, 16)
                    av[i, sl] = av[i, sl] * rb + pv[i, sl] * cb
            return 0
        lax.fori_loop(0, (hi - lo) // 2, row2, 0)

    ad = [None] * BPW
    hd = [None] * BPW
    fd = [None] * BPW
    pd = [None] * BPW
    ad[0], hd[0] = issue_load(0)
    if BPW > 1:
        ad[1], hd[1] = issue_load(1)
    ad[0].wait(); hd[0].wait()
    b1, b2 = issue_init(0)
    pd[0] = issue_par(0)
    b1.wait(); b2.wait()
    c1, c2 = issue_scatter(0)
    c1.wait(); c2.wait()
    d1, d2 = issue_readback(0)
    d1.wait(); d2.wait()

    for q in range(BPW):
        nxt = q + 1 < BPW
        if nxt:
            ad[q + 1].wait(); hd[q + 1].wait()
            b1, b2 = issue_init(q + 1)
            pd[q + 1] = issue_par(q + 1)
        coef_phase(q)
        if nxt:
            b1.wait(); b2.wait()
            c1, c2 = issue_scatter(q + 1)
        if q + 2 < BPW:
            if q >= 1:
                fd[q - 1].wait()
            ad[q + 2], hd[q + 2] = issue_load(q + 2)
        pd[q].wait()
        finalize_rows(q, 0, L // 2)
        if nxt:
            c1.wait(); c2.wait()
            d1, d2 = issue_readback(q + 1)
        finalize_rows(q, L // 2, L)
        fd[q] = pltpu.async_copy(fvs[q % 3],
                                 out_hbm.at[pl.ds((base + q) * L, L)],
                                 sFs[q % 3])
        if nxt:
            d1.wait(); d2.wait()

    for q in range(max(0, BPW - 2), BPW):
        fd[q].wait()


def _sc_agg(feats2, head):
    mesh = plsc.VectorSubcoreMesh(core_axis_name="c", subcore_axis_name="s")
    return pl.kernel(
        _sc_agg_body,
        out_type=jax.ShapeDtypeStruct((B_SC * L, D), jnp.float32),
        mesh=mesh,
        compiler_params=pltpu.CompilerParams(needs_layout_passes=False,
                                             use_tc_tiling_on_sc=False),
        scratch_types=[
            pltpu.VMEM((L, D), jnp.float32),
            pltpu.VMEM((L, D), jnp.float32),
            pltpu.VMEM((L, D), jnp.float32),
            pltpu.VMEM((L,), jnp.int32),
            pltpu.VMEM((L,), jnp.int32),
            pltpu.VMEM((L,), jnp.int32),
            pltpu.VMEM((L, D), jnp.float32),
            pltpu.VMEM((L, D), jnp.float32),
            pltpu.VMEM((L, 16), jnp.float32),
            pltpu.VMEM((L, 16), jnp.float32),
            pltpu.VMEM((L,), jnp.int32),
            pltpu.VMEM((L,), jnp.int32),
            pltpu.VMEM((L, 16), jnp.float32),
            pltpu.VMEM((L, 16), jnp.float32),
            pltpu.VMEM((L,), jnp.float32),
            pltpu.VMEM((L,), jnp.float32),
            pltpu.VMEM_SHARED((NS, L, D), jnp.float32),
            pltpu.VMEM_SHARED((NS, L, 16), jnp.float32),
        ] + [pltpu.SemaphoreType.DMA] * 17,
    )(feats2, head)


GB = 16
GA = 8


def _tail(x, w_ref, b_ref, c_ref):
    h = jnp.dot(x, w_ref[...], preferred_element_type=jnp.float32) + b_ref[...]
    h = jnp.maximum(h, 0.0).astype(jnp.bfloat16)
    s = lax.dot_general(h, c_ref[...], (((1,), (1,)), ((), ())),
                        preferred_element_type=jnp.float32)
    mx = jnp.max(s, axis=-1, keepdims=True)
    e = jnp.exp(s - mx)
    return e / jnp.sum(e, axis=-1, keepdims=True)


def _tc_tail_body(x_ref, w_ref, b_ref, c_ref, part_ref, o_ref):
    del part_ref
    x = x_ref[...].astype(jnp.bfloat16)
    o_ref[...] = _tail(x, w_ref, b_ref, c_ref).reshape(GB, L, K)


def _tc_tail(aggn2, W_gnn, b_gnn, centroids, partial_out):
    return pl.pallas_call(
        _tc_tail_body,
        grid=(B_SC // GB,),
        in_specs=[
            pl.BlockSpec((GB * L, D), lambda i: (i, 0)),
            pl.BlockSpec((D, D), lambda i: (0, 0)),
            pl.BlockSpec((1, D), lambda i: (0, 0)),
            pl.BlockSpec((K, D), lambda i: (0, 0)),
            pl.BlockSpec((GB, L, K), lambda i: (0, 0, 0)),
        ],
        out_specs=pl.BlockSpec((GB, L, K), lambda i: (i, 0, 0)),
        out_shape=jax.ShapeDtypeStruct((B, L, K), jnp.float32),
        input_output_aliases={4: 0},
    )(aggn2, W_gnn.astype(jnp.bfloat16), b_gnn.reshape(1, D),
      centroids.astype(jnp.bfloat16), partial_out)


def _tc_full_body(feats_ref, head_ref, w_ref, b_ref, c_ref, o_ref):
    ii = lax.broadcasted_iota(jnp.int32, (L, L), 0)
    jj = lax.broadcasted_iota(jnp.int32, (L, L), 1)
    eye = (ii == jj)
    aggs = []
    for s in range(GA):
        h_row = head_ref[s].reshape(1, L)
        h_col = h_row.reshape(L, 1)
        oh = (h_col == jj)
        ohT = (h_row == ii)
        adj = jnp.minimum(oh.astype(jnp.float32) + ohT.astype(jnp.float32)
                          + eye.astype(jnp.float32), 1.0)
        rdeg = 1.0 / jnp.maximum(jnp.sum(adj, axis=1, keepdims=True), 1.0)
        fs = feats_ref[s].astype(jnp.bfloat16)
        agg = jnp.dot(adj.astype(jnp.bfloat16), fs,
                      preferred_element_type=jnp.float32)
        aggs.append(agg * rdeg)
    x = jnp.concatenate(aggs, axis=0).astype(jnp.bfloat16)
    o_ref[...] = _tail(x, w_ref, b_ref, c_ref).reshape(GA, L, K)


def _tc_full(feats, head, W_gnn, b_gnn, centroids):
    off = B_SC // GA
    return pl.pallas_call(
        _tc_full_body,
        grid=(B_TC // GA,),
        in_specs=[
            pl.BlockSpec((GA, L, D), lambda i: (i + off, 0, 0)),
            pl.BlockSpec((GA, L), lambda i: (i + off, 0)),
            pl.BlockSpec((D, D), lambda i: (0, 0)),
            pl.BlockSpec((1, D), lambda i: (0, 0)),
            pl.BlockSpec((K, D), lambda i: (0, 0)),
        ],
        out_specs=pl.BlockSpec((GA, L, K), lambda i: (i + off, 0, 0)),
        out_shape=jax.ShapeDtypeStruct((B, L, K), jnp.float32),
    )(feats, head, W_gnn.astype(jnp.bfloat16), b_gnn.reshape(1, D),
      centroids.astype(jnp.bfloat16))


def kernel(feats, tokens, aspect, pos, post, head, deprel, sen_len, adk,
           pos_mask, word_mask, aspect_pos_start, aspect_pos_end,
           plain_text, text_list, W_gnn, b_gnn, centroids):
    h32 = head.astype(jnp.int32)
    aggn2 = _sc_agg(feats.reshape(B * L, D), h32)
    out_tc = _tc_full(feats, h32, W_gnn, b_gnn, centroids)
    return _tc_tail(aggn2, W_gnn, b_gnn, centroids, out_tc)
